# baseline (speedup 1.0000x reference)
import functools

import jax
import jax.numpy as jnp
from jax import lax
from jax.experimental import pallas as pl
from jax.experimental.pallas import tpu as pltpu
from jax.experimental.pallas import tpu_sc as plsc

N = 2
LQ = 5440
LEN_IN = 5440
DM = 256
M = 8
L = 4
P = 4
D = 32
SIZES = (64, 32, 16, 8)
LSI = (0, 4096, 5120, 5376)

NQ_TOT = N * LQ
NW = 32
QPW = NQ_TOT // NW
TBL_ROWS = N * LEN_IN * M

RB = 680
QB = 680


def _mm_body(x_ref, w_ref, b_ref, o_ref):
    o_ref[0] = (
        jnp.dot(x_ref[0], w_ref[...], preferred_element_type=jnp.float32)
        + b_ref[0]
    )


def _matmul_bias(x, w, b, rb):
    n, rows, _ = x.shape
    return pl.pallas_call(
        _mm_body,
        grid=(n, rows // rb),
        in_specs=[
            pl.BlockSpec((1, rb, DM), lambda i, r: (i, r, 0)),
            pl.BlockSpec((DM, DM), lambda i, r: (0, 0)),
            pl.BlockSpec((1, DM), lambda i, r: (0, 0)),
        ],
        out_specs=pl.BlockSpec((1, rb, DM), lambda i, r: (i, r, 0)),
        out_shape=jax.ShapeDtypeStruct((n, rows, DM), jnp.float32),
    )(x, w, b.reshape(1, DM))


def _prep_body(q_ref, x_ref, rpx_ref, rpy_ref, wox_ref, box_ref, woy_ref,
               boy_ref, wa_ref, ba_ref, wv_ref, bv_ref, val_ref, idx_ref,
               wts_ref):
    n = pl.program_id(0)
    q = q_ref[0]
    val_ref[0] = (
        jnp.dot(x_ref[0], wv_ref[...], preferred_element_type=jnp.float32)
        + bv_ref[0]
    )
    hp = lax.Precision.HIGHEST
    offx = jnp.dot(q, wox_ref[...], preferred_element_type=jnp.float32,
                   precision=hp) + box_ref[0]
    offy = jnp.dot(q, woy_ref[...], preferred_element_type=jnp.float32,
                   precision=hp) + boy_ref[0]
    logits = jnp.dot(q, wa_ref[...], preferred_element_type=jnp.float32,
                     precision=hp) + ba_ref[0]

    lane = lax.broadcasted_iota(jnp.int32, (1, 128), 1)
    grp_row = lax.broadcasted_iota(jnp.int32, (128, 1), 0) // 16
    seg = (grp_row == (lane // 16)).astype(jnp.float32)
    e = jnp.exp(logits)
    den = jnp.dot(e, seg, preferred_element_type=jnp.float32)
    attn = e / den

    lvl = (lane // 4) % 4
    s = jnp.where(lvl == 0, 64.0,
        jnp.where(lvl == 1, 32.0,
        jnp.where(lvl == 2, 16.0, 8.0)))
    lsi = jnp.where(lvl == 0, 0.0,
          jnp.where(lvl == 1, 4096.0,
          jnp.where(lvl == 2, 5120.0, 5376.0)))
    head = (lane // 16).astype(jnp.float32)

    lvl4 = lax.broadcasted_iota(jnp.int32, (4, 1), 0)
    eb = (lvl4 == lvl).astype(jnp.float32)
    rx = jnp.dot(rpx_ref[0], eb, preferred_element_type=jnp.float32,
                 precision=hp)
    ry = jnp.dot(rpy_ref[0], eb, preferred_element_type=jnp.float32,
                 precision=hp)

    x = rx * s + offx - 0.5
    y = ry * s + offy - 0.5
    x0 = jnp.floor(x)
    y0 = jnp.floor(y)
    fx = x - x0
    fy = y - y0

    base = (jnp.float32(n) * jnp.float32(LEN_IN) + lsi)
    in_mid = (x0 >= 0.0) & (x0 <= s - 2.0)
    wl = jnp.where(in_mid, 1.0 - fx,
                   jnp.where(x0 == -1.0, fx, 0.0))
    wr = jnp.where(in_mid, fx,
                   jnp.where(x0 == s - 1.0, 1.0 - fx, 0.0))
    sx = jnp.clip(x0, 0.0, s - 2.0)
    for cy in (0, 1):
        yi = y0 + cy
        validy = (yi >= 0.0) & (yi <= s - 1.0)
        ycc = jnp.clip(yi, 0.0, s - 1.0)
        wy = (fy if cy == 1 else (1.0 - fy)) * attn
        wy = wy * validy.astype(jnp.float32)
        idxf = (base + ycc * s + sx) * 8.0 + head
        idx_ref[0, :, cy, :] = idxf.astype(jnp.int32)
        wts_ref[0, :, cy * 2, :] = wl * wy
        wts_ref[0, :, cy * 2 + 1, :] = wr * wy


def _prep(query, x, rpx, rpy, wox, box, woy, boy, wa, ba, wv, bv):
    return pl.pallas_call(
        _prep_body,
        grid=(N, LQ // QB),
        in_specs=[
            pl.BlockSpec((1, QB, DM), lambda n, r: (n, r, 0)),
            pl.BlockSpec((1, QB, DM), lambda n, r: (n, r, 0)),
            pl.BlockSpec((1, QB, L), lambda n, r: (n, r, 0)),
            pl.BlockSpec((1, QB, L), lambda n, r: (n, r, 0)),
            pl.BlockSpec((DM, 128), lambda n, r: (0, 0)),
            pl.BlockSpec((1, 128), lambda n, r: (0, 0)),
            pl.BlockSpec((DM, 128), lambda n, r: (0, 0)),
            pl.BlockSpec((1, 128), lambda n, r: (0, 0)),
            pl.BlockSpec((DM, 128), lambda n, r: (0, 0)),
            pl.BlockSpec((1, 128), lambda n, r: (0, 0)),
            pl.BlockSpec((DM, DM), lambda n, r: (0, 0)),
            pl.BlockSpec((1, DM), lambda n, r: (0, 0)),
        ],
        out_specs=[
            pl.BlockSpec((1, QB, DM), lambda n, r: (n, r, 0)),
            pl.BlockSpec((1, QB, 2, 128), lambda n, r: (n, r, 0, 0)),
            pl.BlockSpec((1, QB, 4, 128), lambda n, r: (n, r, 0, 0)),
        ],
        out_shape=[
            jax.ShapeDtypeStruct((N, LQ, DM), jnp.float32),
            jax.ShapeDtypeStruct((N, LQ, 2, 128), jnp.int32),
            jax.ShapeDtypeStruct((N, LQ, 4, 128), jnp.float32),
        ],
    )(query, x, rpx, rpy, wox, box.reshape(1, 128), woy,
      boy.reshape(1, 128), wa, ba.reshape(1, 128), wv, bv.reshape(1, DM))


def _bcast_lane(vec, lane):
    dn = lax.GatherDimensionNumbers(
        offset_dims=(), collapsed_slice_dims=(0,), start_index_map=(0,))
    return lax.gather(vec, jnp.full((16, 1), lane, jnp.int32), dn, (1,),
                      mode=lax.GatherScatterMode.PROMISE_IN_BOUNDS)


def _sc_body(table_hbm, idx_hbm, wts_hbm, out_hbm, idx_v, wts_v, rows_v,
             out_v, sem_in0, sem_in1, sem_g0, sem_g1, sem_o0, sem_o1):
    wid = lax.axis_index("c") * 16 + lax.axis_index("s")
    q0 = wid * QPW
    sem_in = (sem_in0, sem_in1)
    sem_g = (sem_g0, sem_g1)
    sem_o = (sem_o0, sem_o1)

    def fire_in(i, b):
        pltpu.async_copy(idx_hbm.at[q0 + i], idx_v.at[b], sem_in[b])
        pltpu.async_copy(wts_hbm.at[q0 + i], wts_v.at[b], sem_in[b])

    def wait_in(b):
        pltpu.make_async_copy(idx_hbm.at[0], idx_v.at[b], sem_in[b]).wait()
        pltpu.make_async_copy(wts_hbm.at[0], wts_v.at[b], sem_in[b]).wait()

    def fire_g(b):
        for r in range(2):
            pltpu.async_copy(table_hbm.at[idx_v.at[b, r]], rows_v.at[b, r],
                             sem_g[b])

    def wait_g(b):
        for r in range(2):
            pltpu.make_async_copy(table_hbm.at[pl.ds(0, 128)],
                                  rows_v.at[b, r], sem_g[b]).wait()

    def fire_out(i, b):
        pltpu.async_copy(out_v.at[b], out_hbm.at[pl.ds((q0 + i) * M, M)],
                         sem_o[b])

    def wait_out(b):
        pltpu.make_async_copy(out_v.at[b], out_hbm.at[pl.ds(0, M)],
                              sem_o[b]).wait()

    def compute(b):
        def mstep(mh, carry):
            for dm in range(2):
                m = mh * 2 + dm
                acc0 = jnp.zeros((16,), jnp.float32)
                acc1 = jnp.zeros((16,), jnp.float32)
                for r in range(2):
                    wrow_l = wts_v[b, r * 2, pl.ds(m * 16, 16)]
                    wrow_r = wts_v[b, r * 2 + 1, pl.ds(m * 16, 16)]
                    for pp in range(16):
                        j = m * 16 + pp
                        wvl = _bcast_lane(wrow_l, pp)
                        wvr = _bcast_lane(wrow_r, pp)
                        acc0 = (acc0 + wvl * rows_v[b, r, j, 0:16]
                                + wvr * rows_v[b, r, j, 32:48])
                        acc1 = (acc1 + wvl * rows_v[b, r, j, 16:32]
                                + wvr * rows_v[b, r, j, 48:64])
                out_v[b, m, 0:16] = acc0
                out_v[b, m, 16:32] = acc1
            return carry

        lax.fori_loop(0, M // 2, mstep, 0)

    def steady(i, b, nb, fire_next_in, fire_next_g, do_wait_out):
        wait_in(nb)
        if fire_next_g:
            fire_g(nb)
        wait_g(b)
        if do_wait_out:
            wait_out(b)
        compute(b)
        fire_out(i, b)
        if fire_next_in:
            fire_in(i + 2, b)

    fire_in(0, 0)
    wait_in(0)
    fire_g(0)
    fire_in(1, 1)
    steady(0, 0, 1, True, True, False)
    steady(1, 1, 0, True, True, False)

    def body(k, carry):
        i = 2 * k
        steady(i, 0, 1, True, True, True)
        steady(i + 1, 1, 0, True, True, True)
        return carry

    lax.fori_loop(1, QPW // 2 - 1, body, 0)

    i = QPW - 2
    steady(i, 0, 1, False, True, True)
    wait_g(1)
    wait_out(1)
    compute(1)
    fire_out(i + 1, 1)
    wait_out(0)
    wait_out(1)


@functools.cache
def _get_sc_sample():
    return pl.kernel(
        _sc_body,
        out_type=jax.ShapeDtypeStruct((NQ_TOT * M, D), jnp.float32),
        mesh=plsc.VectorSubcoreMesh(core_axis_name="c", subcore_axis_name="s"),
        compiler_params=pltpu.CompilerParams(use_tc_tiling_on_sc=False),
        scratch_types=[
            pltpu.VMEM((2, 2, 128), jnp.int32),
            pltpu.VMEM((2, 4, 128), jnp.float32),
            pltpu.VMEM((2, 2, 128, 2 * D), jnp.float32),
            pltpu.VMEM((2, M, D), jnp.float32),
            pltpu.SemaphoreType.DMA,
            pltpu.SemaphoreType.DMA,
            pltpu.SemaphoreType.DMA,
            pltpu.SemaphoreType.DMA,
            pltpu.SemaphoreType.DMA,
            pltpu.SemaphoreType.DMA,
        ],
    )


def kernel(query, reference_points, input_flatten, input_spatial_shapes,
           input_level_start_index, W_off, b_off, W_attn, b_attn,
           W_value, b_value, W_out, b_out):
    del input_spatial_shapes, input_level_start_index
    wox = W_off.reshape(DM, 128, 2)[:, :, 0]
    woy = W_off.reshape(DM, 128, 2)[:, :, 1]
    box = b_off.reshape(128, 2)[:, 0]
    boy = b_off.reshape(128, 2)[:, 1]
    rpx = reference_points[..., 0]
    rpy = reference_points[..., 1]

    value, idx4, wts4 = _prep(query, input_flatten, rpx, rpy, wox, box,
                              woy, boy, W_attn, b_attn, W_value, b_value)
    vf = value.reshape(TBL_ROWS, D)
    vn = jnp.concatenate([vf[M:], jnp.zeros((M, D), jnp.float32)], axis=0)
    table = jnp.concatenate([vf, vn], axis=1)
    sc_out = _get_sc_sample()(table, idx4.reshape(NQ_TOT, 2, 128),
                              wts4.reshape(NQ_TOT, 4, 128))
    out = _matmul_bias(sc_out.reshape(N, LQ, DM), W_out, b_out, RB)
    return out


# speedup vs baseline: 1.1113x; 1.1113x over previous
import functools

import jax
import jax.numpy as jnp
from jax import lax
from jax.experimental import pallas as pl
from jax.experimental.pallas import tpu as pltpu
from jax.experimental.pallas import tpu_sc as plsc

N = 2
LQ = 5440
LEN_IN = 5440
DM = 256
M = 8
L = 4
P = 4
D = 32
SIZES = (64, 32, 16, 8)
LSI = (0, 4096, 5120, 5376)

NQ_TOT = N * LQ
NW = 32
QPW = NQ_TOT // NW
TBL_ROWS = N * LEN_IN * M

RB = 680
QB = 680


def _mm_body(x_ref, w_ref, b_ref, o_ref):
    o_ref[0] = (
        jnp.dot(x_ref[0], w_ref[...], preferred_element_type=jnp.float32)
        + b_ref[0]
    )


def _matmul_bias(x, w, b, rb):
    n, rows, _ = x.shape
    return pl.pallas_call(
        _mm_body,
        grid=(n, rows // rb),
        in_specs=[
            pl.BlockSpec((1, rb, DM), lambda i, r: (i, r, 0)),
            pl.BlockSpec((DM, DM), lambda i, r: (0, 0)),
            pl.BlockSpec((1, DM), lambda i, r: (0, 0)),
        ],
        out_specs=pl.BlockSpec((1, rb, DM), lambda i, r: (i, r, 0)),
        out_shape=jax.ShapeDtypeStruct((n, rows, DM), jnp.float32),
    )(x, w, b.reshape(1, DM))


def _prep_body(q_ref, x_ref, rpx_ref, rpy_ref, wox_ref, box_ref, woy_ref,
               boy_ref, wa_ref, ba_ref, wv_ref, bv_ref, val_ref, idx_ref,
               wts_ref):
    n = pl.program_id(0)
    q = q_ref[0]
    val_ref[0] = (
        jnp.dot(x_ref[0], wv_ref[...], preferred_element_type=jnp.float32)
        + bv_ref[0]
    )
    hp = lax.Precision.HIGHEST
    offx = jnp.dot(q, wox_ref[...], preferred_element_type=jnp.float32,
                   precision=hp) + box_ref[0]
    offy = jnp.dot(q, woy_ref[...], preferred_element_type=jnp.float32,
                   precision=hp) + boy_ref[0]
    logits = jnp.dot(q, wa_ref[...], preferred_element_type=jnp.float32,
                     precision=hp) + ba_ref[0]

    lane = lax.broadcasted_iota(jnp.int32, (1, 128), 1)
    grp_row = lax.broadcasted_iota(jnp.int32, (128, 1), 0) // 16
    seg = (grp_row == (lane // 16)).astype(jnp.float32)
    e = jnp.exp(logits)
    den = jnp.dot(e, seg, preferred_element_type=jnp.float32)
    attn = e / den

    lvl = (lane // 4) % 4
    s = jnp.where(lvl == 0, 64.0,
        jnp.where(lvl == 1, 32.0,
        jnp.where(lvl == 2, 16.0, 8.0)))
    lsi = jnp.where(lvl == 0, 0.0,
          jnp.where(lvl == 1, 4096.0,
          jnp.where(lvl == 2, 5120.0, 5376.0)))
    head = (lane // 16).astype(jnp.float32)

    lvl4 = lax.broadcasted_iota(jnp.int32, (4, 1), 0)
    eb = (lvl4 == lvl).astype(jnp.float32)
    rx = jnp.dot(rpx_ref[0], eb, preferred_element_type=jnp.float32,
                 precision=hp)
    ry = jnp.dot(rpy_ref[0], eb, preferred_element_type=jnp.float32,
                 precision=hp)

    x = rx * s + offx - 0.5
    y = ry * s + offy - 0.5
    x0 = jnp.floor(x)
    y0 = jnp.floor(y)
    fx = x - x0
    fy = y - y0

    base = (jnp.float32(n) * jnp.float32(LEN_IN) + lsi)
    for c, (cx, cy) in enumerate(((0.0, 0.0), (0.0, 1.0), (1.0, 0.0), (1.0, 1.0))):
        xi = x0 + cx
        yi = y0 + cy
        valid = ((xi >= 0.0) & (xi <= s - 1.0)
                 & (yi >= 0.0) & (yi <= s - 1.0))
        xc = jnp.clip(xi, 0.0, s - 1.0)
        yc = jnp.clip(yi, 0.0, s - 1.0)
        wx = fx if cx == 1.0 else (1.0 - fx)
        wy = fy if cy == 1.0 else (1.0 - fy)
        wgt = wx * wy * attn * valid.astype(jnp.float32)
        idxf = (base + yc * s + xc) * 8.0 + head
        idx_ref[0, :, c, :] = idxf.astype(jnp.int32)
        wts_ref[0, :, c, :] = wgt


def _prep(query, x, rpx, rpy, wox, box, woy, boy, wa, ba, wv, bv):
    return pl.pallas_call(
        _prep_body,
        grid=(N, LQ // QB),
        in_specs=[
            pl.BlockSpec((1, QB, DM), lambda n, r: (n, r, 0)),
            pl.BlockSpec((1, QB, DM), lambda n, r: (n, r, 0)),
            pl.BlockSpec((1, QB, L), lambda n, r: (n, r, 0)),
            pl.BlockSpec((1, QB, L), lambda n, r: (n, r, 0)),
            pl.BlockSpec((DM, 128), lambda n, r: (0, 0)),
            pl.BlockSpec((1, 128), lambda n, r: (0, 0)),
            pl.BlockSpec((DM, 128), lambda n, r: (0, 0)),
            pl.BlockSpec((1, 128), lambda n, r: (0, 0)),
            pl.BlockSpec((DM, 128), lambda n, r: (0, 0)),
            pl.BlockSpec((1, 128), lambda n, r: (0, 0)),
            pl.BlockSpec((DM, DM), lambda n, r: (0, 0)),
            pl.BlockSpec((1, DM), lambda n, r: (0, 0)),
        ],
        out_specs=[
            pl.BlockSpec((1, QB, DM), lambda n, r: (n, r, 0)),
            pl.BlockSpec((1, QB, 4, 128), lambda n, r: (n, r, 0, 0)),
            pl.BlockSpec((1, QB, 4, 128), lambda n, r: (n, r, 0, 0)),
        ],
        out_shape=[
            jax.ShapeDtypeStruct((N, LQ, DM), jnp.float32),
            jax.ShapeDtypeStruct((N, LQ, 4, 128), jnp.int32),
            jax.ShapeDtypeStruct((N, LQ, 4, 128), jnp.float32),
        ],
    )(query, x, rpx, rpy, wox, box.reshape(1, 128), woy,
      boy.reshape(1, 128), wa, ba.reshape(1, 128), wv, bv.reshape(1, DM))


def _bcast_lane(vec, lane):
    dn = lax.GatherDimensionNumbers(
        offset_dims=(), collapsed_slice_dims=(0,), start_index_map=(0,))
    return lax.gather(vec, jnp.full((16, 1), lane, jnp.int32), dn, (1,),
                      mode=lax.GatherScatterMode.PROMISE_IN_BOUNDS)


def _sc_body(table_hbm, idx_hbm, wts_hbm, out_hbm, idx_v, wts_v, rows_v,
             out_v, sem_in0, sem_in1, sem_g0, sem_g1, sem_o0, sem_o1):
    wid = lax.axis_index("c") * 16 + lax.axis_index("s")
    q0 = wid * QPW
    sem_in = (sem_in0, sem_in1)
    sem_g = (sem_g0, sem_g1)
    sem_o = (sem_o0, sem_o1)

    def fire_in(i, b):
        pltpu.async_copy(idx_hbm.at[q0 + i], idx_v.at[b], sem_in[b])
        pltpu.async_copy(wts_hbm.at[q0 + i], wts_v.at[b], sem_in[b])

    def wait_in(b):
        pltpu.make_async_copy(idx_hbm.at[0], idx_v.at[b], sem_in[b]).wait()
        pltpu.make_async_copy(wts_hbm.at[0], wts_v.at[b], sem_in[b]).wait()

    def fire_g(b):
        for c in range(4):
            pltpu.async_copy(table_hbm.at[idx_v.at[b, c]], rows_v.at[b, c],
                             sem_g[b])

    def wait_g(b):
        for c in range(4):
            pltpu.make_async_copy(table_hbm.at[pl.ds(0, 128)],
                                  rows_v.at[b, c], sem_g[b]).wait()

    def fire_out(i, b):
        pltpu.async_copy(out_v.at[b], out_hbm.at[pl.ds((q0 + i) * M, M)],
                         sem_o[b])

    def wait_out(b):
        pltpu.make_async_copy(out_v.at[b], out_hbm.at[pl.ds(0, M)],
                              sem_o[b]).wait()

    def compute(b):
        def mstep(mh, carry):
            for dm in range(2):
                m = mh * 2 + dm
                acc0 = jnp.zeros((16,), jnp.float32)
                acc1 = jnp.zeros((16,), jnp.float32)
                for c in range(4):
                    wrow = wts_v[b, c, pl.ds(m * 16, 16)]
                    for pp in range(16):
                        wv = _bcast_lane(wrow, pp)
                        acc0 = acc0 + wv * rows_v[b, c, m * 16 + pp, 0:16]
                        acc1 = acc1 + wv * rows_v[b, c, m * 16 + pp, 16:32]
                out_v[b, m, 0:16] = acc0
                out_v[b, m, 16:32] = acc1
            return carry

        lax.fori_loop(0, M // 2, mstep, 0)

    def steady(i, b, nb, fire_next_in, fire_next_g, do_wait_out):
        wait_in(nb)
        if fire_next_g:
            fire_g(nb)
        wait_g(b)
        if do_wait_out:
            wait_out(b)
        compute(b)
        fire_out(i, b)
        if fire_next_in:
            fire_in(i + 2, b)

    fire_in(0, 0)
    wait_in(0)
    fire_g(0)
    fire_in(1, 1)
    steady(0, 0, 1, True, True, False)
    steady(1, 1, 0, True, True, False)

    def body(k, carry):
        i = 2 * k
        steady(i, 0, 1, True, True, True)
        steady(i + 1, 1, 0, True, True, True)
        return carry

    lax.fori_loop(1, QPW // 2 - 1, body, 0)

    i = QPW - 2
    steady(i, 0, 1, False, True, True)
    wait_g(1)
    wait_out(1)
    compute(1)
    fire_out(i + 1, 1)
    wait_out(0)
    wait_out(1)


@functools.cache
def _get_sc_sample():
    return pl.kernel(
        _sc_body,
        out_type=jax.ShapeDtypeStruct((NQ_TOT * M, D), jnp.float32),
        mesh=plsc.VectorSubcoreMesh(core_axis_name="c", subcore_axis_name="s"),
        compiler_params=pltpu.CompilerParams(use_tc_tiling_on_sc=False),
        scratch_types=[
            pltpu.VMEM((2, 4, 128), jnp.int32),
            pltpu.VMEM((2, 4, 128), jnp.float32),
            pltpu.VMEM((2, 4, 128, D), jnp.float32),
            pltpu.VMEM((2, M, D), jnp.float32),
            pltpu.SemaphoreType.DMA,
            pltpu.SemaphoreType.DMA,
            pltpu.SemaphoreType.DMA,
            pltpu.SemaphoreType.DMA,
            pltpu.SemaphoreType.DMA,
            pltpu.SemaphoreType.DMA,
        ],
    )


def kernel(query, reference_points, input_flatten, input_spatial_shapes,
           input_level_start_index, W_off, b_off, W_attn, b_attn,
           W_value, b_value, W_out, b_out):
    del input_spatial_shapes, input_level_start_index
    wox = W_off.reshape(DM, 128, 2)[:, :, 0]
    woy = W_off.reshape(DM, 128, 2)[:, :, 1]
    box = b_off.reshape(128, 2)[:, 0]
    boy = b_off.reshape(128, 2)[:, 1]
    rpx = reference_points[..., 0]
    rpy = reference_points[..., 1]

    value, idx4, wts4 = _prep(query, input_flatten, rpx, rpy, wox, box,
                              woy, boy, W_attn, b_attn, W_value, b_value)
    table = value.reshape(TBL_ROWS, D)
    sc_out = _get_sc_sample()(table, idx4.reshape(NQ_TOT, 4, 128),
                              wts4.reshape(NQ_TOT, 4, 128))
    out = _matmul_bias(sc_out.reshape(N, LQ, DM), W_out, b_out, RB)
    return out
